# Initial kernel scaffold; baseline (speedup 1.0000x reference)
import jax
import jax.numpy as jnp
from jax.experimental import pallas as pl


def kernel(x, edge_index, batch, fc_W, fc_b, W1, b1, W2, b2):
    raise NotImplementedError("write your pallas kernel here")


# speedup vs baseline: 12.5161x; 12.5161x over previous
import functools

import jax
import jax.numpy as jnp
from jax import lax
from jax.experimental import pallas as pl
from jax.experimental.pallas import tpu as pltpu
from jax.experimental.pallas import tpu_sc as plsc

N = 10000
E = 320000
D = 128
G = 64

NC = 2
NS = 16
NW = NC * NS
EPW = E // NW
K = 80
CHUNKS = EPW // K
NP = 10240
RPT = NP // NS
ZR = 128
DEGW = 128

_mesh = plsc.VectorSubcoreMesh(
    core_axis_name="c", subcore_axis_name="s", num_cores=NC, num_subcores=NS)


def _fill_const(ref, rows, width, val):
    def body(i, _):
        r = i // (width // 16)
        c = (i % (width // 16)) * 16
        ref[r, pl.ds(c, 16)] = jnp.full((16,), val, jnp.float32)
        return 0
    lax.fori_loop(0, rows * (width // 16), body, 0)


_DEG_KERNEL_ARGS = dict(
    out_type=jax.ShapeDtypeStruct((NC * NP, DEGW), jnp.float32),
    mesh=_mesh,
    scratch_types=[
        pltpu.VMEM((K,), jnp.int32),
        pltpu.VMEM((K, DEGW), jnp.float32),
        pltpu.VMEM((ZR, DEGW), jnp.float32),
        pltpu.VMEM_SHARED((NP, DEGW), jnp.float32),
    ],
)


def _deg_body(col_hbm, out_hbm, idx_v, ones_v, zbuf_v, acc_sh):
    cid = lax.axis_index("c")
    sid = lax.axis_index("s")

    _fill_const(zbuf_v, ZR, DEGW, 0.0)
    _fill_const(ones_v, K, DEGW, 1.0)

    for z in range(RPT // ZR):
        pltpu.sync_copy(zbuf_v, acc_sh.at[pl.ds(sid * RPT + z * ZR, ZR)])
    plsc.subcore_barrier()

    ebase = (cid * NS + sid) * EPW

    def body(g, _):
        pltpu.sync_copy(col_hbm.at[pl.ds(ebase + g * K, K)], idx_v)
        pltpu.sync_copy(ones_v, acc_sh.at[idx_v], add=True)
        return 0
    lax.fori_loop(0, CHUNKS, body, 0)

    plsc.subcore_barrier()
    pltpu.sync_copy(acc_sh.at[pl.ds(sid * RPT, RPT)],
                    out_hbm.at[pl.ds(cid * NP + sid * RPT, RPT)])


_deg_kernel = pl.kernel(_deg_body, **_DEG_KERNEL_ARGS)


_CONV_KERNEL_ARGS = dict(
    out_type=jax.ShapeDtypeStruct((NC * NP, D), jnp.float32),
    mesh=_mesh,
    scratch_types=[
        pltpu.VMEM((K,), jnp.int32),
        pltpu.VMEM((K,), jnp.int32),
        pltpu.VMEM((K, D), jnp.float32),
        pltpu.VMEM((ZR, D), jnp.float32),
        pltpu.VMEM_SHARED((NP, D), jnp.float32),
        pltpu.SemaphoreType.DMA,
    ],
)


def _conv_body(u_hbm, row_hbm, col_hbm, out_hbm,
               row_v, col_v, rows_v, zbuf_v, acc_sh, sem):
    cid = lax.axis_index("c")
    sid = lax.axis_index("s")

    _fill_const(zbuf_v, ZR, D, 0.0)
    for z in range(RPT // ZR):
        pltpu.sync_copy(zbuf_v, acc_sh.at[pl.ds(sid * RPT + z * ZR, ZR)])
    plsc.subcore_barrier()

    ebase = (cid * NS + sid) * EPW

    def body(g, _):
        pltpu.sync_copy(row_hbm.at[pl.ds(ebase + g * K, K)], row_v)
        pltpu.sync_copy(col_hbm.at[pl.ds(ebase + g * K, K)], col_v)
        pltpu.async_copy(u_hbm.at[row_v], rows_v, sem).wait()
        pltpu.sync_copy(rows_v, acc_sh.at[col_v], add=True)
        return 0
    lax.fori_loop(0, CHUNKS, body, 0)

    plsc.subcore_barrier()
    pltpu.sync_copy(acc_sh.at[pl.ds(sid * RPT, RPT)],
                    out_hbm.at[pl.ds(cid * NP + sid * RPT, RPT)])


_conv_kernel = pl.kernel(_conv_body, **_CONV_KERNEL_ARGS)


def _tc1_body(x_ref, fcwt_ref, fcb_ref, w1t_ref, degp_ref, u1_ref, dis_ref):
    deg = degp_ref[0, 0:N, 0:1] + degp_ref[1, 0:N, 0:1] + 1.0
    dis = lax.rsqrt(deg)
    dis_ref[...] = dis
    h0 = jnp.dot(x_ref[...], fcwt_ref[...],
                 preferred_element_type=jnp.float32) + fcb_ref[...]
    t1 = jnp.dot(h0, w1t_ref[...], preferred_element_type=jnp.float32)
    u1_ref[...] = t1 * dis


def _tc2_body(acc_ref, u_ref, dis_ref, b_ref, wt_ref, uo_ref):
    s = acc_ref[0, 0:N, :] + acc_ref[1, 0:N, :] + u_ref[...]
    dis = dis_ref[...]
    h = jnp.maximum(s * dis + b_ref[...], 0.0)
    t = jnp.dot(h, wt_ref[...], preferred_element_type=jnp.float32)
    uo_ref[...] = t * dis


def _tc3_body(acc_ref, u_ref, dis_ref, b_ref, batch_ref, out_ref):
    s = acc_ref[0, 0:N, :] + acc_ref[1, 0:N, :] + u_ref[...]
    h = jnp.maximum(s * dis_ref[...] + b_ref[...], 0.0)
    seg = lax.broadcasted_iota(jnp.int32, (G, N), 0)
    onehot = (seg == jnp.broadcast_to(batch_ref[...], (G, N))
              ).astype(jnp.float32)
    sums = jnp.dot(onehot, h, preferred_element_type=jnp.float32)
    counts = jnp.sum(onehot, axis=1, keepdims=True)
    out_ref[...] = sums / jnp.maximum(counts, 1.0)


_tc1 = pl.pallas_call(
    _tc1_body,
    out_shape=(jax.ShapeDtypeStruct((N, D), jnp.float32),
               jax.ShapeDtypeStruct((N, 1), jnp.float32)))

_tc2 = pl.pallas_call(
    _tc2_body,
    out_shape=jax.ShapeDtypeStruct((N, D), jnp.float32))

_tc3 = pl.pallas_call(
    _tc3_body,
    out_shape=jax.ShapeDtypeStruct((G, D), jnp.float32))


def kernel(x, edge_index, batch, fc_W, fc_b, W1, b1, W2, b2):
    row = edge_index[0]
    col = edge_index[1]

    degp = _deg_kernel(col).reshape(NC, NP, DEGW)

    u1, dis = _tc1(x, fc_W.T, fc_b.reshape(1, D), W1.T, degp)

    acc1 = _conv_kernel(u1, row, col).reshape(NC, NP, D)
    u2 = _tc2(acc1, u1, dis, b1.reshape(1, D), W2.T)

    acc2 = _conv_kernel(u2, row, col).reshape(NC, NP, D)
    out = _tc3(acc2, u2, dis, b2.reshape(1, D), batch.reshape(1, N))
    return out
